# baseline (speedup 1.0000x reference)
import functools

import jax
import jax.numpy as jnp
from jax import lax
from jax.experimental import pallas as pl
from jax.experimental.pallas import tpu as pltpu
from jax.experimental.pallas import tpu_sc as plsc

N_NODES = 10000
N_EDGES = 320000
D = 128

NC = 2
NS = 16
NW = NC * NS

BATCH = 128
NB = 80
NT = NB // 4
EPW = NB * BATCH
TOTAL = EPW * NW

ROWS_PER_TILE = 632
AGG_ROWS = ROWS_PER_TILE * NS


@functools.partial(
    pl.kernel,
    out_type=jax.ShapeDtypeStruct((NC, AGG_ROWS, D), jnp.float32),
    mesh=plsc.VectorSubcoreMesh(core_axis_name="c", subcore_axis_name="s"),
    scratch_types=[
        [pltpu.VMEM((BATCH,), jnp.int32) for _ in range(4)],
        [pltpu.VMEM((BATCH,), jnp.int32) for _ in range(4)],
        [pltpu.VMEM((BATCH, D), jnp.float32) for _ in range(2)],
        pltpu.VMEM_SHARED((AGG_ROWS, D), jnp.float32),
        [pltpu.SemaphoreType.DMA for _ in range(4)],
        pltpu.SemaphoreType.DMA,
        [pltpu.SemaphoreType.DMA for _ in range(2)],
    ],
)
def _sc_agg(feat_hbm, src_hbm, dst_hbm, zeros_hbm, out_hbm,
            src_v, dst_v, rows_v, agg_sh, sem_i, sem_g, sem_s):
    c = lax.axis_index("c")
    s = lax.axis_index("s")
    wid = c * NS + s

    base = s * ROWS_PER_TILE
    pltpu.sync_copy(zeros_hbm.at[pl.ds(base, ROWS_PER_TILE)],
                    agg_sh.at[pl.ds(base, ROWS_PER_TILE)])
    plsc.subcore_barrier()

    def idx_start(b, k):
        off = wid * EPW + b * BATCH
        pltpu.async_copy(src_hbm.at[pl.ds(off, BATCH)], src_v[k], sem_i[k])
        pltpu.async_copy(dst_hbm.at[pl.ds(off, BATCH)], dst_v[k], sem_i[k])

    def idx_wait(k):
        pltpu.make_async_copy(src_hbm.at[pl.ds(0, BATCH)], src_v[k],
                              sem_i[k]).wait()
        pltpu.make_async_copy(dst_hbm.at[pl.ds(0, BATCH)], dst_v[k],
                              sem_i[k]).wait()

    def scatter_wait(r, k):
        pltpu.make_async_copy(rows_v[r], agg_sh.at[dst_v[k]], sem_s[r]).wait()

    idx_start(0, 0)
    idx_start(1, 1)

    def body(t, carry):
        b4 = 4 * t
        for j in range(4):
            b = b4 + j
            r = j & 1
            k2 = (j + 2) % 4

            if j >= 2:
                scatter_wait(r, k2)
            else:
                @pl.when(t > 0)
                def _():
                    scatter_wait(r, k2)

            if j < 2:
                idx_start(b + 2, k2)
            else:
                @pl.when(t < NT - 1)
                def _():
                    idx_start(b + 2, k2)

            idx_wait(j)
            pltpu.async_copy(feat_hbm.at[src_v[j]], rows_v[r], sem_g)
            pltpu.make_async_copy(feat_hbm.at[src_v[j]], rows_v[r],
                                  sem_g).wait()
            pltpu.async_copy(rows_v[r], agg_sh.at[dst_v[j]], sem_s[r],
                             add=True)
        return carry

    lax.fori_loop(0, NT, body, 0)
    scatter_wait(0, 2)
    scatter_wait(1, 3)
    plsc.subcore_barrier()

    pltpu.sync_copy(agg_sh.at[pl.ds(base, ROWS_PER_TILE)],
                    out_hbm.at[c, pl.ds(base, ROWS_PER_TILE)])


_BN = 2000


def _tc_body(feat_ref, p_ref, w1_ref, w2_ref, out_ref):
    agg = p_ref[0] + p_ref[1]
    out_ref[...] = (
        jnp.dot(feat_ref[...], w1_ref[...], preferred_element_type=jnp.float32)
        + jnp.dot(agg, w2_ref[...], preferred_element_type=jnp.float32))


def _tc_matmul(feat, partials, W1, W2):
    return pl.pallas_call(
        _tc_body,
        grid=(N_NODES // _BN,),
        in_specs=[
            pl.BlockSpec((_BN, D), lambda i: (i, 0)),
            pl.BlockSpec((NC, _BN, D), lambda i: (0, i, 0)),
            pl.BlockSpec((D, D), lambda i: (0, 0)),
            pl.BlockSpec((D, D), lambda i: (0, 0)),
        ],
        out_specs=pl.BlockSpec((_BN, D), lambda i: (i, 0)),
        out_shape=jax.ShapeDtypeStruct((N_NODES, D), jnp.float32),
    )(feat, partials, W1, W2)


def kernel(feat, edge_index, W1, W2):
    ei = edge_index.astype(jnp.int32)
    pad = TOTAL - N_EDGES
    src = jnp.concatenate([ei[0], jnp.zeros((pad,), jnp.int32)])
    dst = jnp.concatenate([ei[1], jnp.full((pad,), N_NODES, jnp.int32)])
    zeros = jnp.zeros((AGG_ROWS, D), jnp.float32)
    partials = _sc_agg(feat, src, dst, zeros)
    return _tc_matmul(feat, partials, W1, W2)
